# Initial kernel scaffold; baseline (speedup 1.0000x reference)
import jax
import jax.numpy as jnp
from jax import lax
from jax.experimental import pallas as pl
from jax.experimental.pallas import tpu as pltpu
from jax.experimental.pallas import tpu_sc as plsc

N_NODES = 10000
N_EDGES = 160000
D = 256
DH = 128
NC, NS, L = 2, 16, 16
NW = NC * NS

_SC_MESH = dict(core_axis_name="c", subcore_axis_name="s",
                num_cores=NC, num_subcores=NS)

E_PER_W = N_EDGES // NW
FULL_VECS = E_PER_W // L
TAIL = E_PER_W - FULL_VECS * L


def _deg_body(ei_hbm, degp_hbm, idx_v, deg_v):
    c = lax.axis_index("c")
    s = lax.axis_index("s")
    w = s * NC + c
    base = w * E_PER_W

    zero = jnp.zeros((L,), jnp.float32)

    def z(i, carry):
        deg_v[pl.ds(i * L, L)] = zero
        return carry

    lax.fori_loop(0, N_NODES // L, z, None)

    pltpu.sync_copy(ei_hbm.at[0, pl.ds(base, E_PER_W)], idx_v)

    ones = jnp.ones((L,), jnp.float32)

    def acc(i, carry):
        idx = idx_v[pl.ds(i * L, L)]
        plsc.addupdate_scatter(deg_v, [idx], ones)
        return carry

    lax.fori_loop(0, FULL_VECS, acc, None)
    idx = idx_v[pl.ds(E_PER_W - L, L)]
    mask = lax.iota(jnp.int32, L) >= (L - TAIL)
    plsc.addupdate_scatter(deg_v, [idx], ones, mask=mask)

    pltpu.sync_copy(deg_v, degp_hbm.at[w])


_deg_kernel = pl.kernel(
    _deg_body,
    out_type=jax.ShapeDtypeStruct((NW, N_NODES), jnp.float32),
    mesh=plsc.VectorSubcoreMesh(**_SC_MESH),
    scratch_types=[
        pltpu.VMEM((E_PER_W,), jnp.int32),
        pltpu.VMEM((N_NODES,), jnp.float32),
    ],
)

R = 1000


def _mm_body(x_ref, w_ref, b_ref, degp_ref, g_ref):
    i = pl.program_id(0)
    h = lax.dot_general(x_ref[...], w_ref[...], (((1,), (1,)), ((), ())),
                        preferred_element_type=jnp.float32)
    h = h + b_ref[...]
    degs = lax.dynamic_slice(degp_ref[...], (0, i * R), (NW, R))
    dis = lax.rsqrt(jnp.sum(degs, axis=0) + 1.0)
    g = h * dis[:, None]
    g_ref[0] = g[:, :DH]
    g_ref[1] = g[:, DH:]


def _mm_call(x, W, b2, degp):
    return pl.pallas_call(
        _mm_body,
        grid=(N_NODES // R,),
        in_specs=[
            pl.BlockSpec((R, D), lambda i: (i, 0)),
            pl.BlockSpec((D, D), lambda i: (0, 0)),
            pl.BlockSpec((1, D), lambda i: (0, 0)),
            pl.BlockSpec((NW, N_NODES), lambda i: (0, 0)),
        ],
        out_specs=pl.BlockSpec((NC, R, DH), lambda i: (0, i, 0)),
        out_shape=jax.ShapeDtypeStruct((NC, N_NODES, DH), jnp.float32),
    )(x, W, b2, degp)

CH = 80
NCHUNK = 125
E_PER_T = CH * NCHUNK
NPT = N_NODES // NS
VPC = CH // L


def _edge_body(g_hbm, ei_hbm, acc_hbm, row_v, col_v, gbuf, acc_sh, sem):
    c = lax.axis_index("c")
    s = lax.axis_index("s")

    pltpu.sync_copy(ei_hbm.at[0, s], row_v)
    pltpu.sync_copy(ei_hbm.at[1, s], col_v)

    off = jnp.full((L,), 1, jnp.int32) * (c * N_NODES)

    def addoff(i, carry):
        k = i // VPC
        j = i % VPC
        r = row_v.at[k]
        r[pl.ds(j * L, L)] = r[pl.ds(j * L, L)] + off
        return carry

    lax.fori_loop(0, NCHUNK * VPC, addoff, None)

    pltpu.sync_copy(g_hbm.at[pl.ds(c * N_NODES + s * NPT, NPT)],
                    acc_sh.at[pl.ds(s * NPT, NPT)])
    plsc.subcore_barrier()

    def chunk(k, carry):
        pltpu.async_copy(g_hbm.at[row_v.at[k]], gbuf, sem).wait()
        pltpu.sync_copy(gbuf, acc_sh.at[col_v.at[k]], add=True)
        return carry

    lax.fori_loop(0, NCHUNK, chunk, None)
    plsc.subcore_barrier()

    pltpu.sync_copy(acc_sh.at[pl.ds(s * NPT, NPT)],
                    acc_hbm.at[pl.ds(c * N_NODES + s * NPT, NPT)])


_edge_kernel = pl.kernel(
    _edge_body,
    out_type=jax.ShapeDtypeStruct((NC * N_NODES, DH), jnp.float32),
    mesh=plsc.VectorSubcoreMesh(**_SC_MESH),
    scratch_types=[
        pltpu.VMEM((NCHUNK, CH), jnp.int32),
        pltpu.VMEM((NCHUNK, CH), jnp.int32),
        pltpu.VMEM((CH, DH), jnp.float32),
        pltpu.VMEM_SHARED((N_NODES, DH), jnp.float32),
        pltpu.SemaphoreType.DMA,
    ],
)


def _fin_body(a0_ref, a1_ref, degp_ref, o_ref):
    i = pl.program_id(0)
    degs = lax.dynamic_slice(degp_ref[...], (0, i * R), (NW, R))
    dis = lax.rsqrt(jnp.sum(degs, axis=0) + 1.0)
    acc = jnp.concatenate([a0_ref[...], a1_ref[...]], axis=1)
    o_ref[...] = jnp.maximum(acc * dis[:, None], 0.0)


def _fin_call(acc, degp):
    return pl.pallas_call(
        _fin_body,
        grid=(N_NODES // R,),
        in_specs=[
            pl.BlockSpec((R, DH), lambda i: (i, 0)),
            pl.BlockSpec((R, DH), lambda i: (i + N_NODES // R, 0)),
            pl.BlockSpec((NW, N_NODES), lambda i: (0, 0)),
        ],
        out_specs=pl.BlockSpec((R, D), lambda i: (i, 0)),
        out_shape=jax.ShapeDtypeStruct((N_NODES, D), jnp.float32),
    )(acc, acc, degp)


def kernel(x, edge_index, W, b):
    ei = edge_index.astype(jnp.int32)
    degp = _deg_kernel(ei)
    g = _mm_call(x, W, b.reshape(1, D), degp)
    g_flat = g.reshape(NC * N_NODES, DH)
    ei_r = ei.reshape(2, NS, NCHUNK, CH)
    acc = _edge_kernel(g_flat, ei_r)
    return _fin_call(acc, degp)


# speedup vs baseline: 14.9289x; 14.9289x over previous
import jax
import jax.numpy as jnp
from jax import lax
from jax.experimental import pallas as pl
from jax.experimental.pallas import tpu as pltpu
from jax.experimental.pallas import tpu_sc as plsc

N_NODES = 10000
N_EDGES = 160000
D = 256
DH = 128
NC, NS, L = 2, 16, 16
NW = NC * NS

_SC_MESH = dict(core_axis_name="c", subcore_axis_name="s",
                num_cores=NC, num_subcores=NS)

E_PER_W = N_EDGES // NW
FULL_VECS = E_PER_W // L
TAIL = E_PER_W - FULL_VECS * L


def _deg_body(row_hbm, degp_hbm, idx_v, deg_v):
    c = lax.axis_index("c")
    s = lax.axis_index("s")
    w = s * NC + c
    base = w * E_PER_W

    zero = jnp.zeros((L,), jnp.float32)

    def z(i, carry):
        deg_v[pl.ds(i * L, L)] = zero
        return carry

    lax.fori_loop(0, N_NODES // L, z, None)

    pltpu.sync_copy(row_hbm.at[pl.ds(base, E_PER_W)], idx_v)

    ones = jnp.ones((L,), jnp.float32)

    def acc(i, carry):
        idx = idx_v[pl.ds(i * L, L)]
        plsc.addupdate_scatter(deg_v, [idx], ones)
        return carry

    lax.fori_loop(0, FULL_VECS, acc, None)
    idx = idx_v[pl.ds(E_PER_W - L, L)]
    mask = lax.iota(jnp.int32, L) >= (L - TAIL)
    plsc.addupdate_scatter(deg_v, [idx], ones, mask=mask)

    pltpu.sync_copy(deg_v, degp_hbm.at[pl.ds(w * N_NODES, N_NODES)])


_deg_kernel = pl.kernel(
    _deg_body,
    out_type=jax.ShapeDtypeStruct((NW * N_NODES,), jnp.float32),
    mesh=plsc.VectorSubcoreMesh(**_SC_MESH),
    scratch_types=[
        pltpu.VMEM((E_PER_W,), jnp.int32),
        pltpu.VMEM((N_NODES,), jnp.float32),
    ],
    compiler_params=pltpu.CompilerParams(needs_layout_passes=False),
)

R = 1000


def _mm_body(x_ref, w_ref, b_ref, degp_ref, g_ref):
    h = lax.dot_general(x_ref[...], w_ref[...], (((1,), (1,)), ((), ())),
                        preferred_element_type=jnp.float32)
    h = h + b_ref[...]
    dis = lax.rsqrt(jnp.sum(degp_ref[...], axis=1) + 1.0)
    g = h * dis[:, None]
    g_ref[0] = g[:, :DH]
    g_ref[1] = g[:, DH:]


def _mm_call(x, W, b2, degp):
    return pl.pallas_call(
        _mm_body,
        grid=(N_NODES // R,),
        in_specs=[
            pl.BlockSpec((R, D), lambda i: (i, 0)),
            pl.BlockSpec((D, D), lambda i: (0, 0)),
            pl.BlockSpec((1, D), lambda i: (0, 0)),
            pl.BlockSpec((R, NW), lambda i: (i, 0)),
        ],
        out_specs=pl.BlockSpec((NC, R, DH), lambda i: (0, i, 0)),
        out_shape=jax.ShapeDtypeStruct((NC, N_NODES, DH), jnp.float32),
    )(x, W, b2, degp)

CH = 80
NCHUNK = 125
E_PER_T = CH * NCHUNK
NPT = 632
NPT_LAST = N_NODES - (NS - 1) * NPT
VPC = CH // L


def _edge_body(g_hbm, row3_hbm, col3_hbm, acc_hbm, row_v, col_v, gbuf, acc_sh, sem):
    c = lax.axis_index("c")
    s = lax.axis_index("s")

    pltpu.sync_copy(row3_hbm.at[s], row_v)
    pltpu.sync_copy(col3_hbm.at[s], col_v)

    off = jnp.full((L,), 1, jnp.int32) * (c * N_NODES)

    def addoff(i, carry):
        k = i // VPC
        j = i % VPC
        r = row_v.at[k]
        r[pl.ds(j * L, L)] = r[pl.ds(j * L, L)] + off
        return carry

    lax.fori_loop(0, NCHUNK * VPC, addoff, None)

    @pl.when(s < NS - 1)
    def _():
        pltpu.sync_copy(g_hbm.at[pl.ds(c * N_NODES + s * NPT, NPT)],
                        acc_sh.at[pl.ds(s * NPT, NPT)])

    @pl.when(s == NS - 1)
    def _():
        pltpu.sync_copy(g_hbm.at[pl.ds(c * N_NODES + s * NPT, NPT_LAST)],
                        acc_sh.at[pl.ds(s * NPT, NPT_LAST)])

    plsc.subcore_barrier()

    def chunk(k, carry):
        pltpu.async_copy(g_hbm.at[row_v.at[k]], gbuf, sem).wait()
        pltpu.sync_copy(gbuf, acc_sh.at[col_v.at[k]], add=True)
        return carry

    lax.fori_loop(0, NCHUNK, chunk, None)
    plsc.subcore_barrier()

    @pl.when(s < NS - 1)
    def _():
        pltpu.sync_copy(acc_sh.at[pl.ds(s * NPT, NPT)],
                        acc_hbm.at[pl.ds(c * N_NODES + s * NPT, NPT)])

    @pl.when(s == NS - 1)
    def _():
        pltpu.sync_copy(acc_sh.at[pl.ds(s * NPT, NPT_LAST)],
                        acc_hbm.at[pl.ds(c * N_NODES + s * NPT, NPT_LAST)])


_edge_kernel = pl.kernel(
    _edge_body,
    out_type=jax.ShapeDtypeStruct((NC * N_NODES, DH), jnp.float32),
    mesh=plsc.VectorSubcoreMesh(**_SC_MESH),
    scratch_types=[
        pltpu.VMEM((NCHUNK, CH), jnp.int32),
        pltpu.VMEM((NCHUNK, CH), jnp.int32),
        pltpu.VMEM((CH, DH), jnp.float32),
        pltpu.VMEM_SHARED((N_NODES, DH), jnp.float32),
        pltpu.SemaphoreType.DMA,
    ],
    compiler_params=pltpu.CompilerParams(needs_layout_passes=False),
)


def _fin_body(a0_ref, a1_ref, degp_ref, o_ref):
    dis = lax.rsqrt(jnp.sum(degp_ref[...], axis=1) + 1.0)
    acc = jnp.concatenate([a0_ref[...], a1_ref[...]], axis=1)
    o_ref[...] = jnp.maximum(acc * dis[:, None], 0.0)


def _fin_call(acc, degp):
    return pl.pallas_call(
        _fin_body,
        grid=(N_NODES // R,),
        in_specs=[
            pl.BlockSpec((R, DH), lambda i: (i, 0)),
            pl.BlockSpec((R, DH), lambda i: (i + N_NODES // R, 0)),
            pl.BlockSpec((R, NW), lambda i: (i, 0)),
        ],
        out_specs=pl.BlockSpec((R, D), lambda i: (i, 0)),
        out_shape=jax.ShapeDtypeStruct((N_NODES, D), jnp.float32),
    )(acc, acc, degp)


def kernel(x, edge_index, W, b):
    ei = edge_index.astype(jnp.int32)
    row = ei[0]
    col = ei[1]
    degp = _deg_kernel(row).reshape(NW, N_NODES).T
    g = _mm_call(x, W, b.reshape(1, D), degp)
    g_flat = g.reshape(NC * N_NODES, DH)
    row3 = row.reshape(NS, NCHUNK, CH)
    col3 = col.reshape(NS, NCHUNK, CH)
    acc = _edge_kernel(g_flat, row3, col3)
    return _fin_call(acc, degp)


# speedup vs baseline: 22.0948x; 1.4800x over previous
import jax
import jax.numpy as jnp
from jax import lax
from jax.experimental import pallas as pl
from jax.experimental.pallas import tpu as pltpu
from jax.experimental.pallas import tpu_sc as plsc

N_NODES = 10000
N_EDGES = 160000
D = 256
DH = 128
NC, NS, L = 2, 16, 16
NW = NC * NS

_SC_MESH = dict(core_axis_name="c", subcore_axis_name="s",
                num_cores=NC, num_subcores=NS)

E_PER_W = N_EDGES // NW
FULL_VECS = E_PER_W // L
TAIL = E_PER_W - FULL_VECS * L


def _deg_body(row_hbm, degp_hbm, idx_v, deg_v):
    c = lax.axis_index("c")
    s = lax.axis_index("s")
    w = s * NC + c
    base = w * E_PER_W

    zero = jnp.zeros((L,), jnp.float32)

    def z(i, carry):
        deg_v[pl.ds(i * L, L)] = zero
        return carry

    lax.fori_loop(0, N_NODES // L, z, None)

    pltpu.sync_copy(row_hbm.at[pl.ds(base, E_PER_W)], idx_v)

    ones = jnp.ones((L,), jnp.float32)

    def acc(i, carry):
        idx = idx_v[pl.ds(i * L, L)]
        plsc.addupdate_scatter(deg_v, [idx], ones)
        return carry

    lax.fori_loop(0, FULL_VECS, acc, None)
    idx = idx_v[pl.ds(E_PER_W - L, L)]
    mask = lax.iota(jnp.int32, L) >= (L - TAIL)
    plsc.addupdate_scatter(deg_v, [idx], ones, mask=mask)

    pltpu.sync_copy(deg_v, degp_hbm.at[pl.ds(w * N_NODES, N_NODES)])


_deg_kernel = pl.kernel(
    _deg_body,
    out_type=jax.ShapeDtypeStruct((NW * N_NODES,), jnp.float32),
    mesh=plsc.VectorSubcoreMesh(**_SC_MESH),
    scratch_types=[
        pltpu.VMEM((E_PER_W,), jnp.int32),
        pltpu.VMEM((N_NODES,), jnp.float32),
    ],
    compiler_params=pltpu.CompilerParams(needs_layout_passes=False),
)

R = 1000


def _mm_body(x_ref, w_ref, b_ref, degp_ref, g_ref):
    h = lax.dot_general(x_ref[...], w_ref[...], (((1,), (1,)), ((), ())),
                        preferred_element_type=jnp.float32)
    h = h + b_ref[...]
    dis = lax.rsqrt(jnp.sum(degp_ref[...], axis=1) + 1.0)
    g = h * dis[:, None]
    g_ref[0] = g[:, :DH]
    g_ref[1] = g[:, DH:]


def _mm_call(x, W, b2, degp):
    return pl.pallas_call(
        _mm_body,
        grid=(N_NODES // R,),
        in_specs=[
            pl.BlockSpec((R, D), lambda i: (i, 0)),
            pl.BlockSpec((D, D), lambda i: (0, 0)),
            pl.BlockSpec((1, D), lambda i: (0, 0)),
            pl.BlockSpec((R, NW), lambda i: (i, 0)),
        ],
        out_specs=pl.BlockSpec((NC, R, DH), lambda i: (0, i, 0)),
        out_shape=jax.ShapeDtypeStruct((NC, N_NODES, DH), jnp.float32),
    )(x, W, b2, degp)

CH = 80
E_PER_T = N_EDGES // NS
NF = E_PER_T // CH
NPT = 632
NPT_LAST = N_NODES - (NS - 1) * NPT
VPC = CH // L


def _edge_body(g_hbm, rowm_hbm, colm_hbm, acc_hbm,
               row_v, col_v, gbuf, acc_sh, gs):
    c = lax.axis_index("c")
    s = lax.axis_index("s")

    pltpu.sync_copy(rowm_hbm.at[pl.ds(s * E_PER_T, E_PER_T)], row_v)
    pltpu.sync_copy(colm_hbm.at[s], col_v)

    off = jnp.full((L,), 1, jnp.int32) * (c * N_NODES)

    def addoff(i, carry):
        row_v[pl.ds(i * L, L)] = row_v[pl.ds(i * L, L)] + off
        return carry

    lax.fori_loop(0, E_PER_T // L, addoff, None)

    @pl.when(s < NS - 1)
    def _():
        pltpu.sync_copy(g_hbm.at[pl.ds(c * N_NODES + s * NPT, NPT)],
                        acc_sh.at[pl.ds(s * NPT, NPT)])

    @pl.when(s == NS - 1)
    def _():
        pltpu.sync_copy(g_hbm.at[pl.ds(c * N_NODES + s * NPT, NPT_LAST)],
                        acc_sh.at[pl.ds(s * NPT, NPT_LAST)])

    plsc.subcore_barrier()

    pltpu.async_copy(g_hbm.at[row_v.at[pl.ds(0, CH)]], gbuf.at[0], gs.at[0])

    def chunk(k, carry):
        b = lax.rem(k, 2)
        bn = lax.rem(k + 1, 2)

        @pl.when(k + 1 < NF)
        def _():
            pltpu.async_copy(g_hbm.at[row_v.at[pl.ds((k + 1) * CH, CH)]],
                             gbuf.at[bn], gs.at[bn])

        pltpu.make_async_copy(g_hbm.at[row_v.at[pl.ds(k * CH, CH)]],
                              gbuf.at[b], gs.at[b]).wait()
        pltpu.sync_copy(gbuf.at[b], acc_sh.at[col_v.at[k]], add=True)
        return carry

    lax.fori_loop(0, NF, chunk, None)
    plsc.subcore_barrier()

    @pl.when(s < NS - 1)
    def _():
        pltpu.sync_copy(acc_sh.at[pl.ds(s * NPT, NPT)],
                        acc_hbm.at[pl.ds(c * N_NODES + s * NPT, NPT)])

    @pl.when(s == NS - 1)
    def _():
        pltpu.sync_copy(acc_sh.at[pl.ds(s * NPT, NPT_LAST)],
                        acc_hbm.at[pl.ds(c * N_NODES + s * NPT, NPT_LAST)])


_edge_kernel = pl.kernel(
    _edge_body,
    out_type=jax.ShapeDtypeStruct((NC * N_NODES, DH), jnp.float32),
    mesh=plsc.VectorSubcoreMesh(**_SC_MESH),
    scratch_types=[
        pltpu.VMEM((E_PER_T,), jnp.int32),
        pltpu.VMEM((NF, CH), jnp.int32),
        pltpu.VMEM((2, CH, DH), jnp.float32),
        pltpu.VMEM_SHARED((N_NODES, DH), jnp.float32),
        pltpu.SemaphoreType.DMA((2,)),
    ],
    compiler_params=pltpu.CompilerParams(needs_layout_passes=False),
)


def _fin_body(a0_ref, a1_ref, degp_ref, o_ref):
    dis = lax.rsqrt(jnp.sum(degp_ref[...], axis=1) + 1.0)
    acc = jnp.concatenate([a0_ref[...], a1_ref[...]], axis=1)
    o_ref[...] = jnp.maximum(acc * dis[:, None], 0.0)


def _fin_call(acc, degp):
    return pl.pallas_call(
        _fin_body,
        grid=(N_NODES // R,),
        in_specs=[
            pl.BlockSpec((R, DH), lambda i: (i, 0)),
            pl.BlockSpec((R, DH), lambda i: (i + N_NODES // R, 0)),
            pl.BlockSpec((R, NW), lambda i: (i, 0)),
        ],
        out_specs=pl.BlockSpec((R, D), lambda i: (i, 0)),
        out_shape=jax.ShapeDtypeStruct((N_NODES, D), jnp.float32),
    )(acc, acc, degp)


def kernel(x, edge_index, W, b):
    ei = edge_index.astype(jnp.int32)
    row = ei[0]
    col = ei[1]
    degp = _deg_kernel(row).reshape(NW, N_NODES).T
    g = _mm_call(x, W, b.reshape(1, D), degp)
    g_flat = g.reshape(NC * N_NODES, DH)
    row_m = row
    col_m = col.reshape(NS, NF, CH)
    acc = _edge_kernel(g_flat, row_m, col_m)
    return _fin_call(acc, degp)


# speedup vs baseline: 22.1050x; 1.0005x over previous
import jax
import jax.numpy as jnp
from jax import lax
from jax.experimental import pallas as pl
from jax.experimental.pallas import tpu as pltpu
from jax.experimental.pallas import tpu_sc as plsc

N_NODES = 10000
N_EDGES = 160000
D = 256
DH = 128
NC, NS, L = 2, 16, 16
NW = NC * NS

_SC_MESH = dict(core_axis_name="c", subcore_axis_name="s",
                num_cores=NC, num_subcores=NS)

E_PER_W = N_EDGES // NW
FULL_VECS = E_PER_W // L
TAIL = E_PER_W - FULL_VECS * L


def _deg_body(row_hbm, degp_hbm, idx_v, deg_v):
    c = lax.axis_index("c")
    s = lax.axis_index("s")
    w = s * NC + c
    base = w * E_PER_W

    zero = jnp.zeros((L,), jnp.float32)

    def z(i, carry):
        deg_v[pl.ds(i * L, L)] = zero
        return carry

    lax.fori_loop(0, N_NODES // L, z, None)

    pltpu.sync_copy(row_hbm.at[pl.ds(base, E_PER_W)], idx_v)

    ones = jnp.ones((L,), jnp.float32)

    def acc(i, carry):
        idx = idx_v[pl.ds(i * L, L)]
        plsc.addupdate_scatter(deg_v, [idx], ones)
        return carry

    lax.fori_loop(0, FULL_VECS, acc, None)
    idx = idx_v[pl.ds(E_PER_W - L, L)]
    mask = lax.iota(jnp.int32, L) >= (L - TAIL)
    plsc.addupdate_scatter(deg_v, [idx], ones, mask=mask)

    pltpu.sync_copy(deg_v, degp_hbm.at[pl.ds(w * N_NODES, N_NODES)])


_deg_kernel = pl.kernel(
    _deg_body,
    out_type=jax.ShapeDtypeStruct((NW * N_NODES,), jnp.float32),
    mesh=plsc.VectorSubcoreMesh(**_SC_MESH),
    scratch_types=[
        pltpu.VMEM((E_PER_W,), jnp.int32),
        pltpu.VMEM((N_NODES,), jnp.float32),
    ],
    compiler_params=pltpu.CompilerParams(needs_layout_passes=False),
)

R = 1000


def _mm_body(x_ref, w_ref, b_ref, degp_ref, g_ref):
    h = lax.dot_general(x_ref[...], w_ref[...], (((1,), (1,)), ((), ())),
                        preferred_element_type=jnp.float32)
    h = h + b_ref[...]
    dis = lax.rsqrt(jnp.sum(degp_ref[...], axis=1) + 1.0)
    g = h * dis[:, None]
    g_ref[0] = g[:, :DH]
    g_ref[1] = g[:, DH:]


def _mm_call(x, W, b2, degp):
    return pl.pallas_call(
        _mm_body,
        grid=(N_NODES // R,),
        in_specs=[
            pl.BlockSpec((R, D), lambda i: (i, 0)),
            pl.BlockSpec((D, D), lambda i: (0, 0)),
            pl.BlockSpec((1, D), lambda i: (0, 0)),
            pl.BlockSpec((R, NW), lambda i: (i, 0)),
        ],
        out_specs=pl.BlockSpec((NC, R, DH), lambda i: (0, i, 0)),
        out_shape=jax.ShapeDtypeStruct((NC, N_NODES, DH), jnp.float32),
    )(x, W, b2, degp)

CH = 80
E_PER_T = N_EDGES // NS
NF = E_PER_T // CH
NPT = 632
NPT_LAST = N_NODES - (NS - 1) * NPT
VPC = CH // L


def _edge_body(g_hbm, rowm_hbm, colm_hbm, acc_hbm,
               row_v, col_v, gbuf, acc_sh, gs, ss):
    c = lax.axis_index("c")
    s = lax.axis_index("s")

    pltpu.sync_copy(rowm_hbm.at[pl.ds(s * E_PER_T, E_PER_T)], row_v)
    pltpu.sync_copy(colm_hbm.at[s], col_v)

    off = jnp.full((L,), 1, jnp.int32) * (c * N_NODES)

    def addoff(i, carry):
        row_v[pl.ds(i * L, L)] = row_v[pl.ds(i * L, L)] + off
        return carry

    lax.fori_loop(0, E_PER_T // L, addoff, None)

    @pl.when(s < NS - 1)
    def _():
        pltpu.sync_copy(g_hbm.at[pl.ds(c * N_NODES + s * NPT, NPT)],
                        acc_sh.at[pl.ds(s * NPT, NPT)])

    @pl.when(s == NS - 1)
    def _():
        pltpu.sync_copy(g_hbm.at[pl.ds(c * N_NODES + s * NPT, NPT_LAST)],
                        acc_sh.at[pl.ds(s * NPT, NPT_LAST)])

    plsc.subcore_barrier()

    pltpu.async_copy(g_hbm.at[row_v.at[pl.ds(0, CH)]], gbuf.at[0], gs.at[0])

    def chunk(k, carry):
        b = lax.rem(k, 2)
        bn = lax.rem(k + 1, 2)

        @pl.when(k >= 1)
        def _():
            pltpu.make_async_copy(gbuf.at[bn], acc_sh.at[col_v.at[k - 1]],
                                  ss.at[bn]).wait()

        @pl.when(k + 1 < NF)
        def _():
            pltpu.async_copy(g_hbm.at[row_v.at[pl.ds((k + 1) * CH, CH)]],
                             gbuf.at[bn], gs.at[bn])

        pltpu.make_async_copy(g_hbm.at[row_v.at[pl.ds(k * CH, CH)]],
                              gbuf.at[b], gs.at[b]).wait()
        pltpu.async_copy(gbuf.at[b], acc_sh.at[col_v.at[k]], ss.at[b],
                         add=True)
        return carry

    lax.fori_loop(0, NF, chunk, None)
    pltpu.make_async_copy(gbuf.at[(NF - 1) % 2], acc_sh.at[col_v.at[NF - 1]],
                          ss.at[(NF - 1) % 2]).wait()
    plsc.subcore_barrier()

    @pl.when(s < NS - 1)
    def _():
        pltpu.sync_copy(acc_sh.at[pl.ds(s * NPT, NPT)],
                        acc_hbm.at[pl.ds(c * N_NODES + s * NPT, NPT)])

    @pl.when(s == NS - 1)
    def _():
        pltpu.sync_copy(acc_sh.at[pl.ds(s * NPT, NPT_LAST)],
                        acc_hbm.at[pl.ds(c * N_NODES + s * NPT, NPT_LAST)])


_edge_kernel = pl.kernel(
    _edge_body,
    out_type=jax.ShapeDtypeStruct((NC * N_NODES, DH), jnp.float32),
    mesh=plsc.VectorSubcoreMesh(**_SC_MESH),
    scratch_types=[
        pltpu.VMEM((E_PER_T,), jnp.int32),
        pltpu.VMEM((NF, CH), jnp.int32),
        pltpu.VMEM((2, CH, DH), jnp.float32),
        pltpu.VMEM_SHARED((N_NODES, DH), jnp.float32),
        pltpu.SemaphoreType.DMA((2,)),
        pltpu.SemaphoreType.DMA((2,)),
    ],
    compiler_params=pltpu.CompilerParams(needs_layout_passes=False),
)


def _fin_body(a0_ref, a1_ref, degp_ref, o_ref):
    dis = lax.rsqrt(jnp.sum(degp_ref[...], axis=1) + 1.0)
    acc = jnp.concatenate([a0_ref[...], a1_ref[...]], axis=1)
    o_ref[...] = jnp.maximum(acc * dis[:, None], 0.0)


def _fin_call(acc, degp):
    return pl.pallas_call(
        _fin_body,
        grid=(N_NODES // R,),
        in_specs=[
            pl.BlockSpec((R, DH), lambda i: (i, 0)),
            pl.BlockSpec((R, DH), lambda i: (i + N_NODES // R, 0)),
            pl.BlockSpec((R, NW), lambda i: (i, 0)),
        ],
        out_specs=pl.BlockSpec((R, D), lambda i: (i, 0)),
        out_shape=jax.ShapeDtypeStruct((N_NODES, D), jnp.float32),
    )(acc, acc, degp)


def kernel(x, edge_index, W, b):
    ei = edge_index.astype(jnp.int32)
    row = ei[0]
    col = ei[1]
    degp = _deg_kernel(row).reshape(NW, N_NODES).T
    g = _mm_call(x, W, b.reshape(1, D), degp)
    g_flat = g.reshape(NC * N_NODES, DH)
    row_m = row
    col_m = col.reshape(NS, NF, CH)
    acc = _edge_kernel(g_flat, row_m, col_m)
    return _fin_call(acc, degp)


# speedup vs baseline: 24.6817x; 1.1166x over previous
import jax
import jax.numpy as jnp
from jax import lax
from jax.experimental import pallas as pl
from jax.experimental.pallas import tpu as pltpu
from jax.experimental.pallas import tpu_sc as plsc

N_NODES = 10000
N_EDGES = 160000
D = 256
DH = 128
NC, NS, L = 2, 16, 16
NW = NC * NS

_SC_MESH = dict(core_axis_name="c", subcore_axis_name="s",
                num_cores=NC, num_subcores=NS)

E_PER_W = N_EDGES // NW
FULL_VECS = E_PER_W // L
TAIL = E_PER_W - FULL_VECS * L


def _deg_body(row_hbm, degp_hbm, idx_v, deg_v):
    c = lax.axis_index("c")
    s = lax.axis_index("s")
    w = s * NC + c
    base = w * E_PER_W

    zero = jnp.zeros((L,), jnp.float32)

    def z(i, carry):
        deg_v[pl.ds(i * L, L)] = zero
        return carry

    lax.fori_loop(0, N_NODES // L, z, None)

    pltpu.sync_copy(row_hbm.at[pl.ds(base, E_PER_W)], idx_v)

    ones = jnp.ones((L,), jnp.float32)

    def acc(i, carry):
        idx = idx_v[pl.ds(i * L, L)]
        plsc.addupdate_scatter(deg_v, [idx], ones)
        return carry

    lax.fori_loop(0, FULL_VECS, acc, None)
    idx = idx_v[pl.ds(E_PER_W - L, L)]
    mask = lax.iota(jnp.int32, L) >= (L - TAIL)
    plsc.addupdate_scatter(deg_v, [idx], ones, mask=mask)

    pltpu.sync_copy(deg_v, degp_hbm.at[pl.ds(w * N_NODES, N_NODES)])


_deg_kernel = pl.kernel(
    _deg_body,
    out_type=jax.ShapeDtypeStruct((NW * N_NODES,), jnp.float32),
    mesh=plsc.VectorSubcoreMesh(**_SC_MESH),
    scratch_types=[
        pltpu.VMEM((E_PER_W,), jnp.int32),
        pltpu.VMEM((N_NODES,), jnp.float32),
    ],
    compiler_params=pltpu.CompilerParams(needs_layout_passes=False),
)

R = 1000


def _mm_body(x_ref, w_ref, b_ref, degp_ref, g_ref):
    h = lax.dot_general(x_ref[...], w_ref[...], (((1,), (1,)), ((), ())),
                        preferred_element_type=jnp.float32)
    h = h + b_ref[...]
    dis = lax.rsqrt(jnp.sum(degp_ref[...], axis=1) + 1.0)
    g = h * dis[:, None]
    g_ref[0] = g[:, :DH]
    g_ref[1] = g[:, DH:]


def _mm_call(x, W, b2, degp):
    return pl.pallas_call(
        _mm_body,
        grid=(N_NODES // R,),
        in_specs=[
            pl.BlockSpec((R, D), lambda i: (i, 0)),
            pl.BlockSpec((D, D), lambda i: (0, 0)),
            pl.BlockSpec((1, D), lambda i: (0, 0)),
            pl.BlockSpec((R, NW), lambda i: (i, 0)),
        ],
        out_specs=pl.BlockSpec((NC, R, DH), lambda i: (0, i, 0)),
        out_shape=jax.ShapeDtypeStruct((NC, N_NODES, DH), jnp.float32),
    )(x, W, b2, degp)

CH = 80
E_PER_T = N_EDGES // NS
NF = E_PER_T // CH
NB = 3
NPT = 632
NPT_LAST = N_NODES - (NS - 1) * NPT


def _edge_body(g_hbm, rowm_hbm, colm_hbm, acc_hbm,
               row_v, col_v, gbuf, acc_sh, gs, ss):
    c = lax.axis_index("c")
    s = lax.axis_index("s")

    pltpu.sync_copy(rowm_hbm.at[pl.ds(s * E_PER_T, E_PER_T)], row_v)
    pltpu.sync_copy(colm_hbm.at[pl.ds(s * E_PER_T, E_PER_T)], col_v)

    off = jnp.full((L,), 1, jnp.int32) * (c * N_NODES)

    def addoff(i, carry):
        row_v[pl.ds(i * L, L)] = row_v[pl.ds(i * L, L)] + off
        return carry

    lax.fori_loop(0, E_PER_T // L, addoff, None)

    @pl.when(s < NS - 1)
    def _():
        pltpu.sync_copy(g_hbm.at[pl.ds(c * N_NODES + s * NPT, NPT)],
                        acc_sh.at[pl.ds(s * NPT, NPT)])

    @pl.when(s == NS - 1)
    def _():
        pltpu.sync_copy(g_hbm.at[pl.ds(c * N_NODES + s * NPT, NPT_LAST)],
                        acc_sh.at[pl.ds(s * NPT, NPT_LAST)])

    plsc.subcore_barrier()

    pltpu.async_copy(g_hbm.at[row_v.at[pl.ds(0, CH)]], gbuf.at[0], gs.at[0])

    def chunk(k, carry):
        b = lax.rem(k, NB)
        bn = lax.rem(k + 1, NB)

        @pl.when(k >= NB - 1)
        def _():
            pltpu.make_async_copy(
                gbuf.at[bn],
                acc_sh.at[col_v.at[pl.ds((k - NB + 1) * CH, CH)]],
                ss.at[bn]).wait()

        @pl.when(k + 1 < NF)
        def _():
            pltpu.async_copy(g_hbm.at[row_v.at[pl.ds((k + 1) * CH, CH)]],
                             gbuf.at[bn], gs.at[bn])

        pltpu.make_async_copy(g_hbm.at[row_v.at[pl.ds(k * CH, CH)]],
                              gbuf.at[b], gs.at[b]).wait()
        pltpu.async_copy(gbuf.at[b], acc_sh.at[col_v.at[pl.ds(k * CH, CH)]],
                         ss.at[b], add=True)
        return carry

    lax.fori_loop(0, NF, chunk, None)
    for kk in range(NF - NB + 1, NF):
        pltpu.make_async_copy(gbuf.at[kk % NB],
                              acc_sh.at[col_v.at[pl.ds(kk * CH, CH)]],
                              ss.at[kk % NB]).wait()
    plsc.subcore_barrier()

    @pl.when(s < NS - 1)
    def _():
        pltpu.sync_copy(acc_sh.at[pl.ds(s * NPT, NPT)],
                        acc_hbm.at[pl.ds(c * N_NODES + s * NPT, NPT)])

    @pl.when(s == NS - 1)
    def _():
        pltpu.sync_copy(acc_sh.at[pl.ds(s * NPT, NPT_LAST)],
                        acc_hbm.at[pl.ds(c * N_NODES + s * NPT, NPT_LAST)])


_edge_kernel = pl.kernel(
    _edge_body,
    out_type=jax.ShapeDtypeStruct((NC * N_NODES, DH), jnp.float32),
    mesh=plsc.VectorSubcoreMesh(**_SC_MESH),
    scratch_types=[
        pltpu.VMEM((E_PER_T,), jnp.int32),
        pltpu.VMEM((E_PER_T,), jnp.int32),
        pltpu.VMEM((NB, CH, DH), jnp.float32),
        pltpu.VMEM_SHARED((N_NODES, DH), jnp.float32),
        pltpu.SemaphoreType.DMA((NB,)),
        pltpu.SemaphoreType.DMA((NB,)),
    ],
    compiler_params=pltpu.CompilerParams(needs_layout_passes=False),
)


def _fin_body(a0_ref, a1_ref, degp_ref, o_ref):
    dis = lax.rsqrt(jnp.sum(degp_ref[...], axis=1) + 1.0)
    acc = jnp.concatenate([a0_ref[...], a1_ref[...]], axis=1)
    o_ref[...] = jnp.maximum(acc * dis[:, None], 0.0)


def _fin_call(acc, degp):
    return pl.pallas_call(
        _fin_body,
        grid=(N_NODES // R,),
        in_specs=[
            pl.BlockSpec((R, DH), lambda i: (i, 0)),
            pl.BlockSpec((R, DH), lambda i: (i + N_NODES // R, 0)),
            pl.BlockSpec((R, NW), lambda i: (i, 0)),
        ],
        out_specs=pl.BlockSpec((R, D), lambda i: (i, 0)),
        out_shape=jax.ShapeDtypeStruct((N_NODES, D), jnp.float32),
    )(acc, acc, degp)


def kernel(x, edge_index, W, b):
    ei = edge_index.astype(jnp.int32)
    row = ei[0]
    col = ei[1]
    degp = _deg_kernel(row).reshape(NW, N_NODES).T
    g = _mm_call(x, W, b.reshape(1, D), degp)
    g_flat = g.reshape(NC * N_NODES, DH)
    acc = _edge_kernel(g_flat, row, col)
    return _fin_call(acc, degp)


# speedup vs baseline: 26.2047x; 1.0617x over previous
import jax
import jax.numpy as jnp
from jax import lax
from jax.experimental import pallas as pl
from jax.experimental.pallas import tpu as pltpu
from jax.experimental.pallas import tpu_sc as plsc

N_NODES = 10000
N_EDGES = 160000
D = 256
DH = 128
NC, NS, L = 2, 16, 16
NW = NC * NS

_SC_MESH = dict(core_axis_name="c", subcore_axis_name="s",
                num_cores=NC, num_subcores=NS)

E_PER_W = N_EDGES // NW
FULL_VECS = E_PER_W // L
TAIL = E_PER_W - FULL_VECS * L


def _deg_body(eif_hbm, degp_hbm, idx_v, deg_v):
    c = lax.axis_index("c")
    s = lax.axis_index("s")
    w = s * NC + c
    base = w * E_PER_W

    zero = jnp.zeros((L,), jnp.float32)

    def z(i, carry):
        deg_v[pl.ds(i * L, L)] = zero
        return carry

    lax.fori_loop(0, N_NODES // L, z, None)

    pltpu.sync_copy(eif_hbm.at[pl.ds(base, E_PER_W)], idx_v)

    ones = jnp.ones((L,), jnp.float32)

    def acc(i, carry):
        idx = idx_v[pl.ds(i * L, L)]
        plsc.addupdate_scatter(deg_v, [idx], ones)
        return carry

    lax.fori_loop(0, FULL_VECS, acc, None)
    idx = idx_v[pl.ds(E_PER_W - L, L)]
    mask = lax.iota(jnp.int32, L) >= (L - TAIL)
    plsc.addupdate_scatter(deg_v, [idx], ones, mask=mask)

    pltpu.sync_copy(deg_v, degp_hbm.at[pl.ds(w * N_NODES, N_NODES)])


_deg_kernel = pl.kernel(
    _deg_body,
    out_type=jax.ShapeDtypeStruct((NW * N_NODES,), jnp.float32),
    mesh=plsc.VectorSubcoreMesh(**_SC_MESH),
    scratch_types=[
        pltpu.VMEM((E_PER_W,), jnp.int32),
        pltpu.VMEM((N_NODES,), jnp.float32),
    ],
    compiler_params=pltpu.CompilerParams(needs_layout_passes=False),
)

R = 1000


def _mm_body(x_ref, w_ref, b_ref, degp_ref, g_ref):
    h = lax.dot_general(x_ref[...], w_ref[...], (((1,), (1,)), ((), ())),
                        preferred_element_type=jnp.float32)
    h = h + b_ref[...]
    dis = lax.rsqrt(jnp.sum(degp_ref[...], axis=1) + 1.0)
    g = h * dis[:, None]
    g_ref[0] = g[:, :DH]
    g_ref[1] = g[:, DH:]


def _mm_call(x, W, b2, degp):
    return pl.pallas_call(
        _mm_body,
        grid=(N_NODES // R,),
        in_specs=[
            pl.BlockSpec((R, D), lambda i: (i, 0)),
            pl.BlockSpec((D, D), lambda i: (0, 0)),
            pl.BlockSpec((1, D), lambda i: (0, 0)),
            pl.BlockSpec((R, NW), lambda i: (i, 0)),
        ],
        out_specs=pl.BlockSpec((NC, R, DH), lambda i: (0, i, 0)),
        out_shape=jax.ShapeDtypeStruct((NC, N_NODES, DH), jnp.float32),
    )(x, W, b2, degp)

CH = 80
E_PER_T = N_EDGES // NS
NF = E_PER_T // CH
NB = 3
NPT = 632
NPT_LAST = N_NODES - (NS - 1) * NPT


def _edge_body(g_hbm, eif_hbm, acc_hbm,
               row_v, col_v, gbuf, acc_sh, gs, ss, isem):
    c = lax.axis_index("c")
    s = lax.axis_index("s")

    @pl.when(s < NS - 1)
    def _():
        pltpu.async_copy(g_hbm.at[pl.ds(c * N_NODES + s * NPT, NPT)],
                         acc_sh.at[pl.ds(s * NPT, NPT)], isem.at[0])

    @pl.when(s == NS - 1)
    def _():
        pltpu.async_copy(g_hbm.at[pl.ds(c * N_NODES + s * NPT, NPT_LAST)],
                         acc_sh.at[pl.ds(s * NPT, NPT_LAST)], isem.at[0])

    pltpu.sync_copy(eif_hbm.at[pl.ds(s * E_PER_T, E_PER_T)], row_v)
    pltpu.sync_copy(eif_hbm.at[pl.ds(N_EDGES + s * E_PER_T, E_PER_T)], col_v)

    off = jnp.full((L,), 1, jnp.int32) * (c * N_NODES)

    def addoff(i, carry):
        row_v[pl.ds(i * L, L)] = row_v[pl.ds(i * L, L)] + off
        return carry

    lax.fori_loop(0, E_PER_T // L, addoff, None)

    @pl.when(s < NS - 1)
    def _():
        pltpu.make_async_copy(g_hbm.at[pl.ds(c * N_NODES + s * NPT, NPT)],
                              acc_sh.at[pl.ds(s * NPT, NPT)], isem.at[0]).wait()

    @pl.when(s == NS - 1)
    def _():
        pltpu.make_async_copy(g_hbm.at[pl.ds(c * N_NODES + s * NPT, NPT_LAST)],
                              acc_sh.at[pl.ds(s * NPT, NPT_LAST)],
                              isem.at[0]).wait()

    plsc.subcore_barrier()

    pltpu.async_copy(g_hbm.at[row_v.at[pl.ds(0, CH)]], gbuf.at[0], gs.at[0])

    def chunk(k, carry):
        b = lax.rem(k, NB)
        bn = lax.rem(k + 1, NB)

        @pl.when(k >= NB - 1)
        def _():
            pltpu.make_async_copy(
                gbuf.at[bn],
                acc_sh.at[col_v.at[pl.ds((k - NB + 1) * CH, CH)]],
                ss.at[bn]).wait()

        @pl.when(k + 1 < NF)
        def _():
            pltpu.async_copy(g_hbm.at[row_v.at[pl.ds((k + 1) * CH, CH)]],
                             gbuf.at[bn], gs.at[bn])

        pltpu.make_async_copy(g_hbm.at[row_v.at[pl.ds(k * CH, CH)]],
                              gbuf.at[b], gs.at[b]).wait()
        pltpu.async_copy(gbuf.at[b], acc_sh.at[col_v.at[pl.ds(k * CH, CH)]],
                         ss.at[b], add=True)
        return carry

    lax.fori_loop(0, NF, chunk, None)
    for kk in range(NF - NB + 1, NF):
        pltpu.make_async_copy(gbuf.at[kk % NB],
                              acc_sh.at[col_v.at[pl.ds(kk * CH, CH)]],
                              ss.at[kk % NB]).wait()
    plsc.subcore_barrier()

    @pl.when(s < NS - 1)
    def _():
        pltpu.sync_copy(acc_sh.at[pl.ds(s * NPT, NPT)],
                        acc_hbm.at[pl.ds(c * N_NODES + s * NPT, NPT)])

    @pl.when(s == NS - 1)
    def _():
        pltpu.sync_copy(acc_sh.at[pl.ds(s * NPT, NPT_LAST)],
                        acc_hbm.at[pl.ds(c * N_NODES + s * NPT, NPT_LAST)])


_edge_kernel = pl.kernel(
    _edge_body,
    out_type=jax.ShapeDtypeStruct((NC * N_NODES, DH), jnp.float32),
    mesh=plsc.VectorSubcoreMesh(**_SC_MESH),
    scratch_types=[
        pltpu.VMEM((E_PER_T,), jnp.int32),
        pltpu.VMEM((E_PER_T,), jnp.int32),
        pltpu.VMEM((NB, CH, DH), jnp.float32),
        pltpu.VMEM_SHARED((N_NODES, DH), jnp.float32),
        pltpu.SemaphoreType.DMA((NB,)),
        pltpu.SemaphoreType.DMA((NB,)),
        pltpu.SemaphoreType.DMA((1,)),
    ],
    compiler_params=pltpu.CompilerParams(needs_layout_passes=False),
)


def _fin_body(a0_ref, a1_ref, degp_ref, o_ref):
    dis = lax.rsqrt(jnp.sum(degp_ref[...], axis=1) + 1.0)
    acc = jnp.concatenate([a0_ref[...], a1_ref[...]], axis=1)
    o_ref[...] = jnp.maximum(acc * dis[:, None], 0.0)


def _fin_call(acc, degp):
    return pl.pallas_call(
        _fin_body,
        grid=(N_NODES // R,),
        in_specs=[
            pl.BlockSpec((R, DH), lambda i: (i, 0)),
            pl.BlockSpec((R, DH), lambda i: (i + N_NODES // R, 0)),
            pl.BlockSpec((R, NW), lambda i: (i, 0)),
        ],
        out_specs=pl.BlockSpec((R, D), lambda i: (i, 0)),
        out_shape=jax.ShapeDtypeStruct((N_NODES, D), jnp.float32),
    )(acc, acc, degp)


def kernel(x, edge_index, W, b):
    eif = edge_index.astype(jnp.int32).reshape(2 * N_EDGES)
    degp = _deg_kernel(eif).reshape(NW, N_NODES).T
    g = _mm_call(x, W, b.reshape(1, D), degp)
    g_flat = g.reshape(NC * N_NODES, DH)
    acc = _edge_kernel(g_flat, eif)
    return _fin_call(acc, degp)
